# baseline (speedup 1.0000x reference)
import functools

import jax
import jax.numpy as jnp
from jax import lax
from jax.experimental import pallas as pl
from jax.experimental.pallas import tpu as pltpu
from jax.experimental.pallas import tpu_sc as plsc

NC = 2
NS = 16
NW = NC * NS
B = 24576
BPW = B // NW


def _make_gather_grouped(Dp: int, R: int):
    nchunks = BPW // R
    ngroups = R // 6
    mesh = plsc.VectorSubcoreMesh(core_axis_name="c", subcore_axis_name="s")

    @functools.partial(
        pl.kernel,
        out_type=jax.ShapeDtypeStruct((4096 * 8, Dp), jnp.float32),
        mesh=mesh,
        scratch_types=[
            pltpu.VMEM((R,), jnp.int32),
            pltpu.VMEM((R, Dp), jnp.float32),
            pltpu.SemaphoreType.DMA,
        ],
        compiler_params=pltpu.CompilerParams(use_tc_tiling_on_sc=False),
    )
    def k(idx_hbm, table_hbm, out_hbm, idx_c, rows_v, sem):
        wid = lax.axis_index("s") * NC + lax.axis_index("c")
        base = wid * BPW

        def body(c, carry):
            off = base + pl.multiple_of(c * R, 24)
            pltpu.sync_copy(idx_hbm.at[pl.ds(off, R)], idx_c)
            pltpu.async_copy(table_hbm.at[idx_c], rows_v, sem).wait()
            g0 = off // 6
            for k_ in range(ngroups):
                pltpu.sync_copy(
                    rows_v.at[pl.ds(6 * k_, 6)],
                    out_hbm.at[pl.ds(8 * (g0 + k_), 6)],
                )
            return carry

        lax.fori_loop(0, nchunks, body, 0)

    return k


def _make_gather_flat(Dp: int, R: int):
    nchunks = BPW // R
    mesh = plsc.VectorSubcoreMesh(core_axis_name="c", subcore_axis_name="s")

    @functools.partial(
        pl.kernel,
        out_type=jax.ShapeDtypeStruct((B, Dp), jnp.float32),
        mesh=mesh,
        scratch_types=[
            pltpu.VMEM((R,), jnp.int32),
            pltpu.VMEM((R, Dp), jnp.float32),
            pltpu.SemaphoreType.DMA,
        ],
        compiler_params=pltpu.CompilerParams(use_tc_tiling_on_sc=False),
    )
    def k(idx_hbm, table_hbm, out_hbm, idx_c, rows_v, sem):
        wid = lax.axis_index("s") * NC + lax.axis_index("c")
        base = wid * BPW

        def body(c, carry):
            off = base + pl.multiple_of(c * R, 8)
            pltpu.sync_copy(idx_hbm.at[pl.ds(off, R)], idx_c)
            pltpu.async_copy(table_hbm.at[idx_c], rows_v, sem).wait()
            pltpu.sync_copy(rows_v, out_hbm.at[pl.ds(off, R)])
            return carry

        lax.fori_loop(0, nchunks, body, 0)

    return k


def _pad8(d: int) -> int:
    return (d + 7) // 8 * 8


def _make_depad_tc(D: int, Dp: int, GI: int):
    def body(in_ref, out_ref):
        out_ref[...] = in_ref[...][:, :6, :D]

    return pl.pallas_call(
        body,
        grid=(4096 // GI,),
        in_specs=[pl.BlockSpec((GI, 8, Dp), lambda i: (i, 0, 0))],
        out_specs=pl.BlockSpec((GI, 6, D), lambda i: (i, 0, 0)),
        out_shape=jax.ShapeDtypeStruct((4096, 6, D), jnp.float32),
    )


_CHUNK = {2047: 24, 1023: 48, 511: 96}
_KERNELS = {D: _make_gather_grouped(_pad8(D), R) for D, R in _CHUNK.items()}
_DEPAD_TC = {2047: _make_depad_tc(2047, 2048, 64), 1023: _make_depad_tc(1023, 1024, 64)}
_TERA = _make_gather_flat(_pad8(19), 128)


def kernel(x, species, abilities, items, movesets, teratypes):
    idx = x.reshape(-1).astype(jnp.int32)
    outs = []
    for table in (species, abilities, items, movesets):
        D = table.shape[1]
        Dp = _pad8(D)
        tp = table if Dp == D else jnp.pad(table, ((0, 0), (0, Dp - D)))
        out = _KERNELS[D](idx, tp).reshape(4096, 8, Dp)
        if D in _DEPAD_TC:
            outs.append(_DEPAD_TC[D](out))
        else:
            outs.append(lax.slice(out, (0, 0, 0), (4096, 6, D)))
    tp = jnp.pad(teratypes, ((0, 0), (0, _pad8(19) - 19)))
    out = _TERA(idx, tp)
    outs.append(out[:, :19].reshape(x.shape[0], x.shape[1], 19))
    return (outs[0], outs[1], outs[2], outs[3], outs[4])


# speedup vs baseline: 1.1206x; 1.1206x over previous
import functools

import jax
import jax.numpy as jnp
from jax import lax
from jax.experimental import pallas as pl
from jax.experimental.pallas import tpu as pltpu
from jax.experimental.pallas import tpu_sc as plsc

NC = 2
NS = 16
NW = NC * NS
B = 24576
BPW = B // NW

_DS = (2047, 511, 511, 1023, 19)
_DPS = (2048, 512, 512, 1024, 24)
_RS = (24, 96, 96, 48, 96)

_mesh = plsc.VectorSubcoreMesh(core_axis_name="c", subcore_axis_name="s")


@functools.partial(
    pl.kernel,
    out_type=tuple(
        jax.ShapeDtypeStruct((4096 * 8, dp), jnp.float32) for dp in _DPS[:4]
    )
    + (jax.ShapeDtypeStruct((B, _DPS[4]), jnp.float32),),
    mesh=_mesh,
    scratch_types=[pltpu.SemaphoreType.DMA],
    compiler_params=pltpu.CompilerParams(use_tc_tiling_on_sc=False),
)
def _gather_all(idx_hbm, sp, ab, it, mv, te, o_sp, o_ab, o_it, o_mv, o_te, sem):
    wid = lax.axis_index("s") * NC + lax.axis_index("c")
    base = wid * BPW

    def grouped_phase(table_hbm, out_hbm, Dp, R):
        nchunks = BPW // R
        ngroups = R // 6

        def scoped(idx_c, rows_v):
            def body(c, carry):
                off = base + pl.multiple_of(c * R, 24)
                pltpu.sync_copy(idx_hbm.at[pl.ds(off, R)], idx_c)
                pltpu.async_copy(table_hbm.at[idx_c], rows_v, sem).wait()
                g0 = off // 6
                for k_ in range(ngroups):
                    pltpu.sync_copy(
                        rows_v.at[pl.ds(6 * k_, 6)],
                        out_hbm.at[pl.ds(8 * (g0 + k_), 6)],
                    )
                return carry

            lax.fori_loop(0, nchunks, body, 0)

        pl.run_scoped(
            scoped,
            pltpu.VMEM((R,), jnp.int32),
            pltpu.VMEM((R, Dp), jnp.float32),
        )

    def flat_phase(table_hbm, out_hbm, Dp, R):
        nchunks = BPW // R

        def scoped(idx_c, rows_v):
            def body(c, carry):
                off = base + pl.multiple_of(c * R, 8)
                pltpu.sync_copy(idx_hbm.at[pl.ds(off, R)], idx_c)
                pltpu.async_copy(table_hbm.at[idx_c], rows_v, sem).wait()
                pltpu.sync_copy(rows_v, out_hbm.at[pl.ds(off, R)])
                return carry

            lax.fori_loop(0, nchunks, body, 0)

        pl.run_scoped(
            scoped,
            pltpu.VMEM((R,), jnp.int32),
            pltpu.VMEM((R, Dp), jnp.float32),
        )

    grouped_phase(sp, o_sp, _DPS[0], _RS[0])
    grouped_phase(ab, o_ab, _DPS[1], _RS[1])
    grouped_phase(it, o_it, _DPS[2], _RS[2])
    grouped_phase(mv, o_mv, _DPS[3], _RS[3])
    flat_phase(te, o_te, _DPS[4], _RS[4])


def kernel(x, species, abilities, items, movesets, teratypes):
    idx = x.reshape(-1).astype(jnp.int32)
    tables = (species, abilities, items, movesets, teratypes)
    padded = tuple(
        t if t.shape[1] == dp else jnp.pad(t, ((0, 0), (0, dp - t.shape[1])))
        for t, dp in zip(tables, _DPS)
    )
    o_sp, o_ab, o_it, o_mv, o_te = _gather_all(idx, *padded)
    outs = []
    for o, d, dp in zip((o_sp, o_ab, o_it, o_mv), _DS[:4], _DPS[:4]):
        outs.append(lax.slice(o.reshape(4096, 8, dp), (0, 0, 0), (4096, 6, d)))
    outs.append(o_te[:, :19].reshape(x.shape[0], x.shape[1], 19))
    return (outs[0], outs[1], outs[2], outs[3], outs[4])


# speedup vs baseline: 1.3062x; 1.1656x over previous
import functools

import jax
import jax.numpy as jnp
from jax import lax
from jax.experimental import pallas as pl
from jax.experimental.pallas import tpu as pltpu
from jax.experimental.pallas import tpu_sc as plsc

NC = 2
NS = 16
NW = NC * NS
B = 24576
BPW = B // NW


def _make_gather_grouped(Dp: int, R: int):
    nchunks = BPW // R
    ngroups = R // 6
    mesh = plsc.VectorSubcoreMesh(core_axis_name="c", subcore_axis_name="s")

    @functools.partial(
        pl.kernel,
        out_type=jax.ShapeDtypeStruct((4096 * 8, Dp), jnp.float32),
        mesh=mesh,
        scratch_types=[
            pltpu.VMEM((R,), jnp.int32),
            pltpu.VMEM((R, Dp), jnp.float32),
            pltpu.SemaphoreType.DMA,
        ],
        compiler_params=pltpu.CompilerParams(use_tc_tiling_on_sc=False),
    )
    def k(idx_hbm, table_hbm, out_hbm, idx_c, rows_v, sem):
        wid = lax.axis_index("s") * NC + lax.axis_index("c")
        base = wid * BPW

        def body(c, carry):
            off = base + pl.multiple_of(c * R, 24)
            pltpu.sync_copy(idx_hbm.at[pl.ds(off, R)], idx_c)
            pltpu.async_copy(table_hbm.at[idx_c], rows_v, sem).wait()
            g0 = off // 6
            for k_ in range(ngroups):
                pltpu.sync_copy(
                    rows_v.at[pl.ds(6 * k_, 6)],
                    out_hbm.at[pl.ds(8 * (g0 + k_), 6)],
                )
            return carry

        lax.fori_loop(0, nchunks, body, 0)

    return k


def _make_gather_flat(Dp: int, R: int):
    nchunks = BPW // R
    mesh = plsc.VectorSubcoreMesh(core_axis_name="c", subcore_axis_name="s")

    @functools.partial(
        pl.kernel,
        out_type=jax.ShapeDtypeStruct((B, Dp), jnp.float32),
        mesh=mesh,
        scratch_types=[
            pltpu.VMEM((R,), jnp.int32),
            pltpu.VMEM((R, Dp), jnp.float32),
            pltpu.SemaphoreType.DMA,
        ],
        compiler_params=pltpu.CompilerParams(use_tc_tiling_on_sc=False),
    )
    def k(idx_hbm, table_hbm, out_hbm, idx_c, rows_v, sem):
        wid = lax.axis_index("s") * NC + lax.axis_index("c")
        base = wid * BPW

        def body(c, carry):
            off = base + pl.multiple_of(c * R, 8)
            pltpu.sync_copy(idx_hbm.at[pl.ds(off, R)], idx_c)
            pltpu.async_copy(table_hbm.at[idx_c], rows_v, sem).wait()
            pltpu.sync_copy(rows_v, out_hbm.at[pl.ds(off, R)])
            return carry

        lax.fori_loop(0, nchunks, body, 0)

    return k


def _pad8(d: int) -> int:
    return (d + 7) // 8 * 8


_CHUNK = {2047: 48, 1023: 96, 511: 96}
_KERNELS = {D: _make_gather_grouped(_pad8(D), R) for D, R in _CHUNK.items()}
_TERA = _make_gather_flat(_pad8(19), 128)


def kernel(x, species, abilities, items, movesets, teratypes):
    idx = x.reshape(-1).astype(jnp.int32)
    outs = []
    for table in (species, abilities, items, movesets):
        D = table.shape[1]
        Dp = _pad8(D)
        tp = table if Dp == D else jnp.pad(table, ((0, 0), (0, Dp - D)))
        out = _KERNELS[D](idx, tp).reshape(4096, 8, Dp)
        outs.append(lax.slice(out, (0, 0, 0), (4096, 6, D)))
    tp = jnp.pad(teratypes, ((0, 0), (0, _pad8(19) - 19)))
    out = _TERA(idx, tp)
    outs.append(out[:, :19].reshape(x.shape[0], x.shape[1], 19))
    return (outs[0], outs[1], outs[2], outs[3], outs[4])


# speedup vs baseline: 1.3229x; 1.0128x over previous
import functools

import jax
import jax.numpy as jnp
from jax import lax
from jax.experimental import pallas as pl
from jax.experimental.pallas import tpu as pltpu
from jax.experimental.pallas import tpu_sc as plsc

NC = 2
NS = 16
NW = NC * NS
B = 24576
BPW = B // NW


def _make_gather_grouped(Dp: int, R: int):
    nchunks = BPW // R
    ngroups = R // 6
    mesh = plsc.VectorSubcoreMesh(core_axis_name="c", subcore_axis_name="s")

    @functools.partial(
        pl.kernel,
        out_type=jax.ShapeDtypeStruct((4096 * 8, Dp), jnp.float32),
        mesh=mesh,
        scratch_types=[
            pltpu.VMEM((R,), jnp.int32),
            pltpu.VMEM((R, Dp), jnp.float32),
            pltpu.SemaphoreType.DMA,
        ],
        compiler_params=pltpu.CompilerParams(use_tc_tiling_on_sc=False),
    )
    def k(idx_hbm, table_hbm, out_hbm, idx_c, rows_v, sem):
        wid = lax.axis_index("s") * NC + lax.axis_index("c")
        base = wid * BPW

        def body(c, carry):
            off = base + pl.multiple_of(c * R, 24)
            pltpu.sync_copy(idx_hbm.at[pl.ds(off, R)], idx_c)
            pltpu.async_copy(table_hbm.at[idx_c], rows_v, sem).wait()
            g0 = off // 6
            for k_ in range(ngroups):
                pltpu.sync_copy(
                    rows_v.at[pl.ds(6 * k_, 6)],
                    out_hbm.at[pl.ds(8 * (g0 + k_), 6)],
                )
            return carry

        lax.fori_loop(0, nchunks, body, 0)

    return k


def _make_gather_staged(Dp: int, R: int, V: int = 2048):
    nchunks = BPW // R
    ngroups = R // 6
    rows_per_tile = V // NS
    mesh = plsc.VectorSubcoreMesh(core_axis_name="c", subcore_axis_name="s")

    @functools.partial(
        pl.kernel,
        out_type=jax.ShapeDtypeStruct((4096 * 8, Dp), jnp.float32),
        mesh=mesh,
        scratch_types=[
            pltpu.VMEM((R,), jnp.int32),
            pltpu.VMEM((R, Dp), jnp.float32),
            pltpu.SemaphoreType.DMA,
            pltpu.VMEM_SHARED((V, Dp), jnp.float32),
        ],
        compiler_params=pltpu.CompilerParams(use_tc_tiling_on_sc=False),
    )
    def k(idx_hbm, table_hbm, out_hbm, idx_c, rows_v, sem, stage):
        sid = lax.axis_index("s")
        wid = sid * NC + lax.axis_index("c")
        base = wid * BPW
        srow = sid * rows_per_tile
        pltpu.sync_copy(
            table_hbm.at[pl.ds(srow, rows_per_tile)],
            stage.at[pl.ds(srow, rows_per_tile)],
        )
        plsc.subcore_barrier()

        def body(c, carry):
            off = base + pl.multiple_of(c * R, 24)
            pltpu.sync_copy(idx_hbm.at[pl.ds(off, R)], idx_c)
            pltpu.async_copy(stage.at[idx_c], rows_v, sem).wait()
            g0 = off // 6
            for k_ in range(ngroups):
                pltpu.sync_copy(
                    rows_v.at[pl.ds(6 * k_, 6)],
                    out_hbm.at[pl.ds(8 * (g0 + k_), 6)],
                )
            return carry

        lax.fori_loop(0, nchunks, body, 0)

    return k


def _make_gather_flat(Dp: int, R: int):
    nchunks = BPW // R
    mesh = plsc.VectorSubcoreMesh(core_axis_name="c", subcore_axis_name="s")

    @functools.partial(
        pl.kernel,
        out_type=jax.ShapeDtypeStruct((B, Dp), jnp.float32),
        mesh=mesh,
        scratch_types=[
            pltpu.VMEM((R,), jnp.int32),
            pltpu.VMEM((R, Dp), jnp.float32),
            pltpu.SemaphoreType.DMA,
        ],
        compiler_params=pltpu.CompilerParams(use_tc_tiling_on_sc=False),
    )
    def k(idx_hbm, table_hbm, out_hbm, idx_c, rows_v, sem):
        wid = lax.axis_index("s") * NC + lax.axis_index("c")
        base = wid * BPW

        def body(c, carry):
            off = base + pl.multiple_of(c * R, 8)
            pltpu.sync_copy(idx_hbm.at[pl.ds(off, R)], idx_c)
            pltpu.async_copy(table_hbm.at[idx_c], rows_v, sem).wait()
            pltpu.sync_copy(rows_v, out_hbm.at[pl.ds(off, R)])
            return carry

        lax.fori_loop(0, nchunks, body, 0)

    return k


def _pad8(d: int) -> int:
    return (d + 7) // 8 * 8


_KERNELS = {
    2047: _make_gather_grouped(2048, 48),
    1023: _make_gather_grouped(1024, 96),
    511: _make_gather_staged(512, 96),
}
_TERA = _make_gather_flat(_pad8(19), 128)


def kernel(x, species, abilities, items, movesets, teratypes):
    idx = x.reshape(-1).astype(jnp.int32)
    outs = []
    for table in (species, abilities, items, movesets):
        D = table.shape[1]
        Dp = _pad8(D)
        tp = table if Dp == D else jnp.pad(table, ((0, 0), (0, Dp - D)))
        out = _KERNELS[D](idx, tp).reshape(4096, 8, Dp)
        outs.append(lax.slice(out, (0, 0, 0), (4096, 6, D)))
    tp = jnp.pad(teratypes, ((0, 0), (0, _pad8(19) - 19)))
    out = _TERA(idx, tp)
    outs.append(out[:, :19].reshape(x.shape[0], x.shape[1], 19))
    return (outs[0], outs[1], outs[2], outs[3], outs[4])
